# Initial kernel scaffold; baseline (speedup 1.0000x reference)
import jax
import jax.numpy as jnp
from jax.experimental import pallas as pl


def kernel(L_x_, L_edge_index_, L_self_modules_convs_modules_0_modules_lin_parameters_weight_, L_self_modules_convs_modules_0_parameters_bias_, L_self_modules_convs_modules_1_modules_lin_parameters_weight_, L_self_modules_convs_modules_1_parameters_bias_, L_self_modules_convs_modules_2_modules_lin_parameters_weight_, L_self_modules_convs_modules_2_parameters_bias_, L_self_modules_convs_modules_3_modules_lin_parameters_weight_, L_self_modules_convs_modules_3_parameters_bias_, L_self_modules_convs_modules_4_modules_lin_parameters_weight_, L_self_modules_convs_modules_4_parameters_bias_):
    raise NotImplementedError("write your pallas kernel here")


# speedup vs baseline: 16.2916x; 16.2916x over previous
import jax
import jax.numpy as jnp
from jax.experimental import pallas as pl

_N = 1000
_NP = 1024
_E = 100
_EP = 128
_D = 256


def _gcn5_body(ein_ref, eint_ref, x_ref,
               w0, w1, w2, w3, w4,
               b0, b1, b2, b3, b4,
               out_ref):
    ein = ein_ref[...]
    eint = eint_ref[...]
    src_r = ein[0:1, :]
    dst_r = ein[1:2, :]
    val_r = ein[2:3, :]
    src_c = eint[:, 0:1]
    dst_c = eint[:, 1:2]
    val_c = eint[:, 2:3]

    ew_r = (src_r != dst_r) & (val_r == 1)
    ew_c = (src_c != dst_c) & (val_c == 1)

    iota_ne = jax.lax.broadcasted_iota(jnp.int32, (_NP, _EP), 0)
    iota_en = jax.lax.broadcasted_iota(jnp.int32, (_EP, _NP), 1)

    one = jnp.float32(1.0)
    zero = jnp.float32(0.0)
    oh_dst = jnp.where((iota_ne == dst_r) & ew_r, one, zero)
    oh_srcT = jnp.where((iota_en == src_c) & ew_c, one, zero)
    oh_dstT = jnp.where((iota_en == dst_c) & ew_c, one, zero)

    deg = 1.0 + jnp.sum(oh_dst, axis=1, keepdims=True)
    dis = jax.lax.rsqrt(deg)
    dinv = 1.0 / deg

    dn = (((1,), (0,)), ((), ()))
    dis_src = jax.lax.dot_general(oh_srcT, dis, dn,
                                  preferred_element_type=jnp.float32)
    dis_dst = jax.lax.dot_general(oh_dstT, dis, dn,
                                  preferred_element_type=jnp.float32)
    norm_c = dis_src * dis_dst

    ws = (w0, w1, w2, w3, w4)
    bs = (b0, b1, b2, b3, b4)
    x = x_ref[...]
    for i in range(5):
        g = jax.lax.dot_general(oh_srcT, x, dn,
                                preferred_element_type=jnp.float32)
        scat = jax.lax.dot_general(oh_dst, norm_c * g, dn,
                                   preferred_element_type=jnp.float32)
        y = dinv * x + scat
        h = jax.lax.dot_general(y, ws[i][...], dn,
                                preferred_element_type=jnp.float32)
        h = h + bs[i][...]
        x = jnp.maximum(h, 0.0) if i < 4 else h
    out_ref[...] = x


def kernel(L_x_, L_edge_index_,
           L_self_modules_convs_modules_0_modules_lin_parameters_weight_,
           L_self_modules_convs_modules_0_parameters_bias_,
           L_self_modules_convs_modules_1_modules_lin_parameters_weight_,
           L_self_modules_convs_modules_1_parameters_bias_,
           L_self_modules_convs_modules_2_modules_lin_parameters_weight_,
           L_self_modules_convs_modules_2_parameters_bias_,
           L_self_modules_convs_modules_3_modules_lin_parameters_weight_,
           L_self_modules_convs_modules_3_parameters_bias_,
           L_self_modules_convs_modules_4_modules_lin_parameters_weight_,
           L_self_modules_convs_modules_4_parameters_bias_):
    ws = [L_self_modules_convs_modules_0_modules_lin_parameters_weight_,
          L_self_modules_convs_modules_1_modules_lin_parameters_weight_,
          L_self_modules_convs_modules_2_modules_lin_parameters_weight_,
          L_self_modules_convs_modules_3_modules_lin_parameters_weight_,
          L_self_modules_convs_modules_4_modules_lin_parameters_weight_]
    bs = [L_self_modules_convs_modules_0_parameters_bias_,
          L_self_modules_convs_modules_1_parameters_bias_,
          L_self_modules_convs_modules_2_parameters_bias_,
          L_self_modules_convs_modules_3_parameters_bias_,
          L_self_modules_convs_modules_4_parameters_bias_]

    x = jnp.pad(L_x_, ((0, _NP - _N), (0, 0)))
    e = L_edge_index_.astype(jnp.int32)
    e = jnp.pad(e, ((0, 0), (0, _EP - _E)))
    valid = (jnp.arange(_EP, dtype=jnp.int32) < _E).astype(jnp.int32)
    ein = jnp.zeros((8, _EP), jnp.int32)
    ein = ein.at[0, :].set(e[0]).at[1, :].set(e[1]).at[2, :].set(valid)
    eint = jnp.zeros((_EP, 8), jnp.int32)
    eint = eint.at[:, 0].set(e[0]).at[:, 1].set(e[1]).at[:, 2].set(valid)

    wts = [w.T for w in ws]
    brs = [b.reshape(1, _D) for b in bs]

    out = pl.pallas_call(
        _gcn5_body,
        out_shape=jax.ShapeDtypeStruct((_NP, _D), jnp.float32),
    )(ein, eint, x, *wts, *brs)
    return out[:_N]
